# Initial kernel scaffold; baseline (speedup 1.0000x reference)
import jax
import jax.numpy as jnp
from jax.experimental import pallas as pl


def kernel(x, edge_index, edge_weight, W1, b1, g1, be1, W2, b2, g2, be2):
    raise NotImplementedError("write your pallas kernel here")


# speedup vs baseline: 8.5836x; 8.5836x over previous
import functools

import jax
import jax.numpy as jnp
from jax import lax
from jax.experimental import pallas as pl
from jax.experimental.pallas import tpu as pltpu
from jax.experimental.pallas import tpu_sc as plsc

N = 10000
D = 128
E = 320000

NC = 2
NS = 16
L = 16
NW = NC * NS

CHUNK = 128
ITERS = 79
EPT = ITERS * CHUNK
E_PAD = NW * EPT

ROWS_PER_TILE = N // NS
ZROWS = 25

_MESH = plsc.VectorSubcoreMesh(
    core_axis_name="c", subcore_axis_name="s", num_cores=NC, num_subcores=NS
)


def _fast_rsqrt(v):
  i = plsc.bitcast(v, jnp.int32)
  i = jnp.int32(0x5F3759DF) - lax.shift_right_logical(i, 1)
  y = plsc.bitcast(i, jnp.float32)
  for _ in range(3):
    y = y * (1.5 - 0.5 * v * y * y)
  return y


EPS_A = E_PAD // NS
DROWS = 640
DCH = DROWS // CHUNK


@functools.partial(
    pl.kernel,
    out_type=jax.ShapeDtypeStruct((E_PAD,), jnp.float32),
    mesh=_MESH,
    compiler_params=pltpu.CompilerParams(needs_layout_passes=False, use_tc_tiling_on_sc=False),
    scratch_types=[
        pltpu.VMEM((EPS_A,), jnp.int32),
        pltpu.VMEM((EPS_A,), jnp.float32),
        pltpu.VMEM((DROWS * L,), jnp.float32),
        pltpu.VMEM((DROWS, L), jnp.float32),
        pltpu.VMEM((DCH, CHUNK), jnp.int32),
        pltpu.VMEM((EPT,), jnp.int32),
        pltpu.VMEM((EPT,), jnp.int32),
        pltpu.VMEM((EPT,), jnp.float32),
        pltpu.VMEM((EPT,), jnp.float32),
        pltpu.VMEM_SHARED((DROWS, L), jnp.float32),
    ],
)
def _prep_kernel(src_hbm, dst_hbm, ew_hbm, w_hbm,
                 dst_a, ew_a, deg_v, deg2, iot, src_d, dst_d, ew_d, w_d,
                 deg_sh):
  cid = lax.axis_index("c")
  sid = lax.axis_index("s")
  wid = cid * NS + sid

  def zero_body(i, _):
    deg_v[pl.ds(i * L, L)] = jnp.zeros((L,), jnp.float32)
    deg2[i] = jnp.zeros((L,), jnp.float32)
    return 0
  lax.fori_loop(0, DROWS, zero_body, 0)
  for k in range(DCH):
    for j in range(CHUNK // L):
      iot[k, pl.ds(j * L, L)] = lax.iota(jnp.int32, L) + (k * CHUNK + j * L)

  @pl.when(sid == 0)
  def _():
    pltpu.sync_copy(deg2, deg_sh)
  plsc.subcore_barrier()

  pltpu.sync_copy(dst_hbm.at[pl.ds(sid * EPS_A, EPS_A)], dst_a)
  pltpu.sync_copy(ew_hbm.at[pl.ds(sid * EPS_A, EPS_A)], ew_a)

  def deg_body(v, _):
    d = dst_a[pl.ds(v * L, L)]
    e = ew_a[pl.ds(v * L, L)]
    plsc.addupdate_scatter(deg_v, [d], e)
    return 0
  lax.fori_loop(0, EPS_A // L, deg_body, 0)

  def pack_body(i, _):
    deg2[i] = deg_v[pl.ds(i * L, L)]
    return 0
  lax.fori_loop(0, DROWS, pack_body, 0)
  for k in range(DCH):
    pltpu.sync_copy(deg2.at[pl.ds(k * CHUNK, CHUNK)],
                    deg_sh.at[iot.at[k]], add=True)
  plsc.subcore_barrier()

  pltpu.sync_copy(deg_sh, deg2)

  def isd_body(i, _):
    v = deg2[i] + 1e-12
    deg_v[pl.ds(i * L, L)] = _fast_rsqrt(v)
    return 0
  lax.fori_loop(0, DROWS, isd_body, 0)

  base = wid * EPT
  pltpu.sync_copy(src_hbm.at[pl.ds(base, EPT)], src_d)
  pltpu.sync_copy(dst_hbm.at[pl.ds(base, EPT)], dst_d)
  pltpu.sync_copy(ew_hbm.at[pl.ds(base, EPT)], ew_d)

  def w_body(j, _):
    s = src_d[pl.ds(j * L, L)]
    d = dst_d[pl.ds(j * L, L)]
    e = ew_d[pl.ds(j * L, L)]
    ws = plsc.load_gather(deg_v, [s])
    wd = plsc.load_gather(deg_v, [d])
    w_d[pl.ds(j * L, L)] = ws * e * wd
    return 0
  lax.fori_loop(0, EPT // L, w_body, 0)

  pltpu.sync_copy(w_d, w_hbm.at[pl.ds(base, EPT)])


@functools.partial(
    pl.kernel,
    out_type=jax.ShapeDtypeStruct((NC, N, D), jnp.float32),
    mesh=_MESH,
    compiler_params=pltpu.CompilerParams(needs_layout_passes=False, use_tc_tiling_on_sc=False),
    scratch_types=[
        pltpu.VMEM((EPT,), jnp.int32),
        pltpu.VMEM((EPT,), jnp.float32),
        pltpu.VMEM((ITERS, CHUNK), jnp.int32),
        pltpu.VMEM((CHUNK, D), jnp.float32),
        pltpu.VMEM((ZROWS, D), jnp.float32),
        pltpu.VMEM_SHARED((N, D), jnp.float32),
        pltpu.SemaphoreType.DMA,
    ],
)
def _hop_kernel(h_hbm, src_hbm, dst3_hbm, w_hbm, part_hbm,
                src_v, w_v, dst_v, rows, zbuf, acc, sem):
  cid = lax.axis_index("c")
  sid = lax.axis_index("s")
  wid = cid * NS + sid

  def zb_body(r, _):
    for k in range(D // L):
      zbuf[r, pl.ds(k * L, L)] = jnp.zeros((L,), jnp.float32)
    return 0
  lax.fori_loop(0, ZROWS, zb_body, 0)
  for k in range(ROWS_PER_TILE // ZROWS):
    pltpu.sync_copy(zbuf, acc.at[pl.ds(sid * ROWS_PER_TILE + k * ZROWS, ZROWS)])
  plsc.subcore_barrier()

  base = wid * EPT
  pltpu.sync_copy(src_hbm.at[pl.ds(base, EPT)], src_v)
  pltpu.sync_copy(w_hbm.at[pl.ds(base, EPT)], w_v)
  pltpu.sync_copy(dst3_hbm.at[wid], dst_v)

  def hop_body(i, _):
    pltpu.async_copy(h_hbm.at[src_v.at[pl.ds(i * CHUNK, CHUNK)]], rows, sem
                     ).wait()

    def scale_body(e, _):
      wv = plsc.load_gather(w_v, [jnp.broadcast_to(i * CHUNK + e, (L,))])
      for k in range(D // L):
        rows[e, pl.ds(k * L, L)] = rows[e, pl.ds(k * L, L)] * wv
      return 0
    lax.fori_loop(0, CHUNK, scale_body, 0)

    pltpu.sync_copy(rows, acc.at[dst_v.at[i]], add=True)
    return 0
  lax.fori_loop(0, ITERS, hop_body, 0)

  plsc.subcore_barrier()
  pltpu.sync_copy(acc.at[pl.ds(sid * ROWS_PER_TILE, ROWS_PER_TILE)],
                  part_hbm.at[cid, pl.ds(sid * ROWS_PER_TILE, ROWS_PER_TILE)])


def _mlp_body(x_ref, w1_ref, b1_ref, g1_ref, be1_ref,
              w2_ref, b2_ref, g2_ref, be2_ref, o_ref):
  def layer(h, w, b, g, be):
    h = jnp.dot(h, w[...], preferred_element_type=jnp.float32) + b[...][None, :]
    mean = jnp.mean(h, axis=0, keepdims=True)
    var = jnp.mean((h - mean) * (h - mean), axis=0, keepdims=True)
    h = (h - mean) * lax.rsqrt(var + 1e-5) * g[...][None, :] + be[...][None, :]
    return jnp.maximum(h, 0.0)

  h = layer(x_ref[...], w1_ref, b1_ref, g1_ref, be1_ref)
  o_ref[...] = layer(h, w2_ref, b2_ref, g2_ref, be2_ref)


_mlp_call = pl.pallas_call(
    _mlp_body,
    out_shape=jax.ShapeDtypeStruct((N, D), jnp.float32),
)


def _combine_body(p_ref, o_ref):
  o_ref[...] = p_ref[0] + p_ref[1]


_combine_call = pl.pallas_call(
    _combine_body,
    out_shape=jax.ShapeDtypeStruct((N, D), jnp.float32),
)


def kernel(x, edge_index, edge_weight, W1, b1, g1, be1, W2, b2, g2, be2):
  src = edge_index[0]
  dst = edge_index[1]
  pad = E_PAD - E
  src_p = jnp.concatenate([src, jnp.zeros((pad,), jnp.int32)])
  dst_p = jnp.concatenate([dst, jnp.zeros((pad,), jnp.int32)])
  ew_p = jnp.concatenate([edge_weight, jnp.zeros((pad,), jnp.float32)])
  dst3 = dst_p.reshape(NW, ITERS, CHUNK)

  w = _prep_kernel(src_p, dst_p, ew_p)
  h = _mlp_call(x, W1, b1, g1, be1, W2, b2, g2, be2)

  p1 = _hop_kernel(h, src_p, dst3, w)
  h1 = _combine_call(p1)
  p2 = _hop_kernel(h1, src_p, dst3, w)
  return _combine_call(p2)
